# baseline (speedup 1.0000x reference)
import functools

import jax
import jax.numpy as jnp
import numpy as np
from jax import lax
from jax.experimental import pallas as pl
from jax.experimental.pallas import tpu as pltpu
from jax.experimental.pallas import tpu_sc as plsc


def _fps_body(S, N, x_ref, y_ref, z_ref, idx_ref, cx_ref, cy_ref, cz_ref):
    B = x_ref.shape[0]
    x = x_ref[...]
    y = y_ref[...]
    z = z_ref[...]
    lane = lax.broadcasted_iota(jnp.int32, (B, N), 1)
    col = lax.broadcasted_iota(jnp.int32, (B, 128), 1)

    def step(j2, carry):
        dist, far, aidx, acx, acy, acz = carry
        oh = (lane == far).astype(jnp.float32)
        cx = jnp.sum(x * oh, axis=1, keepdims=True)
        cy = jnp.sum(y * oh, axis=1, keepdims=True)
        cz = jnp.sum(z * oh, axis=1, keepdims=True)
        sel = col == j2
        aidx = jnp.where(sel, jnp.broadcast_to(far, sel.shape), aidx)
        acx = jnp.where(sel, jnp.broadcast_to(cx, sel.shape), acx)
        acy = jnp.where(sel, jnp.broadcast_to(cy, sel.shape), acy)
        acz = jnp.where(sel, jnp.broadcast_to(cz, sel.shape), acz)
        xm = x - cx
        ym = y - cy
        zm = z - cz
        d = (xm * xm + ym * ym) + zm * zm
        dist = jnp.minimum(dist, d)
        m = jnp.max(dist, axis=1, keepdims=True)
        cand = jnp.where(dist == m, lane, N)
        far = jnp.min(cand, axis=1, keepdims=True).astype(jnp.int32)
        return dist, far, aidx, acx, acy, acz

    dist = jnp.full((B, N), 1e10, jnp.float32)
    far = jnp.zeros((B, 1), jnp.int32)
    for c in range(S // 128):
        sl = pl.ds(c * 128, 128)
        dist, far, aidx, acx, acy, acz = lax.fori_loop(
            0, 128, step,
            (dist, far, idx_ref[:, sl], cx_ref[:, sl], cy_ref[:, sl],
             cz_ref[:, sl]))
        idx_ref[:, sl] = aidx
        cx_ref[:, sl] = acx
        cy_ref[:, sl] = acy
        cz_ref[:, sl] = acz


def _fps(x, y, z, S):
    B, N = x.shape
    fn = pl.pallas_call(
        functools.partial(_fps_body, S, N),
        out_shape=(
            jax.ShapeDtypeStruct((B, S), jnp.int32),
            jax.ShapeDtypeStruct((B, S), jnp.float32),
            jax.ShapeDtypeStruct((B, S), jnp.float32),
            jax.ShapeDtypeStruct((B, S), jnp.float32),
        ),
    )
    return fn(x, y, z)


def _dist_body(r2, c_ref, xt_ref, p_ref, o_ref):
    c = c_ref[0]
    xt = xt_ref[0]
    sq_c = jnp.sum(c * c, axis=1, keepdims=True)
    x0 = xt[0:1, :]
    x1 = xt[1:2, :]
    x2 = xt[2:3, :]
    sq_x = (x0 * x0 + x1 * x1) + x2 * x2
    g = lax.dot_general(c, xt, (((1,), (0,)), ((), ())),
                        preferred_element_type=jnp.float32)
    d = (sq_c + sq_x) - 2.0 * g
    maskf = jnp.where(d <= r2, 1.0, 0.0).astype(jnp.bfloat16)
    bits = jnp.dot(maskf, p_ref[...], preferred_element_type=jnp.float32)
    o_ref[0] = bits.astype(jnp.int32)


def _dist_bits(cent_pad, xt_pad, r2):
    B, S, _ = cent_pad.shape
    N = xt_pad.shape[2]
    W = N // 16
    p = np.zeros((N, W), np.float32)
    p[np.arange(N), np.arange(N) // 16] = (2.0 ** (np.arange(N) % 16))
    fn = pl.pallas_call(
        functools.partial(_dist_body, r2),
        grid=(B,),
        in_specs=[
            pl.BlockSpec((1, S, 16), lambda b: (b, 0, 0)),
            pl.BlockSpec((1, 16, N), lambda b: (b, 0, 0)),
            pl.BlockSpec((N, W), lambda b: (0, 0)),
        ],
        out_specs=pl.BlockSpec((1, S, W), lambda b: (b, 0, 0)),
        out_shape=jax.ShapeDtypeStruct((B, S, W), jnp.int32),
    )
    return fn(cent_pad, xt_pad, jnp.asarray(p, jnp.bfloat16))


def _sc_selgather(bits, table, *, B, S, N, ns, dpad):
    rows = B * S
    W = N // 16
    info = plsc.get_sparse_core_info()
    nw = info.num_cores * info.num_subcores
    rpw = rows // nw
    sub = 8
    nouter = W // sub
    sb = 128 // ns
    nsb = rpw // sb
    mesh = plsc.VectorSubcoreMesh(core_axis_name="c", subcore_axis_name="s")

    @functools.partial(
        pl.kernel,
        out_type=jax.ShapeDtypeStruct((rows * ns, dpad), jnp.float32),
        mesh=mesh,
        compiler_params=pltpu.CompilerParams(needs_layout_passes=False,
                                             use_tc_tiling_on_sc=False),
        scratch_types=[
            pltpu.VMEM((rpw, W), jnp.int32),
            pltpu.VMEM((192,), jnp.int32),
            pltpu.VMEM((sb * ns,), jnp.int32),
            pltpu.VMEM((sb * ns, dpad), jnp.float32),
            pltpu.SMEM((1,), jnp.int32),
            pltpu.SemaphoreType.DMA,
        ],
    )
    def k(w_hbm, t_hbm, o_hbm, wslab, idxb, isel, rowsv, cnt_r, sem):
        wid = lax.axis_index("s") * info.num_cores + lax.axis_index("c")
        lanes = lax.broadcasted_iota(jnp.int32, (16,), 0)
        zeros = jnp.zeros((16,), jnp.int32)
        r0 = wid * rpw
        pltpu.sync_copy(w_hbm.at[pl.ds(r0, rpw)], wslab)

        def sb_body(i, carry):
            for rl in range(sb):
                rloc = i * sb + rl
                cnt_r[0] = 0

                def outer(c, carry2):
                    @pl.when(cnt_r[0] < ns)
                    def _():
                        cnt = cnt_r[0]
                        for j in range(sub):
                            wj = c * sub + j
                            w16 = plsc.load_gather(
                                wslab, [zeros + rloc, zeros + wj])
                            msk = ((w16 >> lanes) & 1) == 1
                            mi = msk.astype(jnp.int32)
                            pos = cnt + plsc.cumsum(mi) - 1
                            plsc.store_scatter(idxb, [pos], wj * 16 + lanes,
                                               mask=msk)
                            cnt = cnt + jnp.sum(mi)
                        cnt_r[0] = cnt
                    return carry2

                lax.fori_loop(0, nouter, outer, 0)
                cnt = cnt_r[0]
                first = plsc.load_gather(idxb, [zeros])
                base = ((r0 + rloc) // S) * N
                for kk in range(ns // 16):
                    slot = kk * 16 + lanes
                    cur = plsc.load_gather(idxb, [slot])
                    sel = jnp.where(slot < cnt, cur, first) + base
                    isel[rl * ns + kk * 16:rl * ns + (kk + 1) * 16] = sel

            pltpu.async_copy(t_hbm.at[isel], rowsv, sem).wait()
            pltpu.sync_copy(rowsv, o_hbm.at[pl.ds((r0 + i * sb) * ns,
                                                  sb * ns)])
            return carry

        lax.fori_loop(0, nsb, sb_body, 0)

    return k(bits, table)


def _mlp_body(nl, g, has_cent, refs):
    x_ref = refs[0]
    o_ref = refs[-1]
    base = 1 + (1 if has_cent else 0)
    h = x_ref[...]
    R = h.shape[0]
    for l in range(nl):
        w = refs[base + 2 * l][...]
        b = refs[base + 2 * l + 1][...]
        h = jnp.dot(h, w, preferred_element_type=jnp.float32)
        if l == 0 and has_cent:
            q = jnp.dot(refs[1][...], w, preferred_element_type=jnp.float32)
            c0 = h.shape[1]
            h = h.reshape(R // g, g, c0) + (b - q)[:, None, :]
            h = h.reshape(R, c0)
        else:
            h = h + b
        h = jnp.maximum(h, 0.0)
    cout = h.shape[1]
    m = jnp.max(h.reshape(R // g, g, cout), axis=1)
    o_ref[...] = m


def _mlp(xrows, wbs, R, g, cent=None):
    rows, dpad = xrows.shape
    nl = len(wbs)
    cout = wbs[-1][0].shape[1]
    in_specs = [pl.BlockSpec((R, dpad), lambda i: (i, 0))]
    args = [xrows]
    if cent is not None:
        in_specs.append(pl.BlockSpec((R // g, dpad), lambda i: (i, 0)))
        args.append(cent)
    for (w, b) in wbs:
        in_specs.append(pl.BlockSpec(w.shape, lambda i: (0, 0)))
        in_specs.append(pl.BlockSpec(b.shape, lambda i: (0, 0)))
        args.extend([w, b])
    fn = pl.pallas_call(
        lambda *refs: _mlp_body(nl, g, cent is not None, refs),
        grid=(rows // R,),
        in_specs=in_specs,
        out_specs=pl.BlockSpec((R // g, cout), lambda i: (i, 0)),
        out_shape=jax.ShapeDtypeStruct((rows // g, cout), jnp.float32),
    )
    return fn(*args)


_LDIMS = {"sa1": [3, 32, 32, 64], "sa2": [67, 64, 128, 256],
          "sa3": [259, 256, 512, 1024]}
_PADIN = {"sa1": 16, "sa2": 80, "sa3": 384}


def _fold_weights(params, name):
    chs = _LDIMS[name]
    dpad = _PADIN[name]
    out = []
    c = np.float32(np.sqrt(1.0 + 1e-5))
    for l in range(len(chs) - 1):
        w = params[name + "_W" + str(l)]
        b = params[name + "_b" + str(l)]
        gm = params[name + "_g" + str(l)]
        be = params[name + "_be" + str(l)]
        s = gm / c
        wt = (w * s[:, None]).T
        bb = (b * s + be)[None, :]
        if l == 0 and wt.shape[0] != dpad:
            wt = jnp.pad(wt, ((0, dpad - wt.shape[0]), (0, 0)))
        out.append((wt, bb))
    return out


def kernel(xyz, params):
    B, _, N = xyz.shape

    x0, x1, x2 = xyz[:, 0, :], xyz[:, 1, :], xyz[:, 2, :]
    _, cx1, cy1, cz1 = _fps(x0, x1, x2, 512)
    nexyz1 = jnp.stack([cx1, cy1, cz1], axis=-1)
    cent1 = jnp.pad(nexyz1, ((0, 0), (0, 0), (0, 13)))
    xt1 = jnp.pad(xyz, ((0, 0), (0, 13), (0, 0)))
    w1 = _dist_bits(cent1, xt1, 16.0).reshape(B * 512, N // 16)
    table1 = jnp.pad(jnp.transpose(xyz, (0, 2, 1)),
                     ((0, 0), (0, 0), (0, 13))).reshape(B * N, 16)
    g1 = _sc_selgather(w1, table1, B=B, S=512, N=N, ns=32, dpad=16)

    _, cx2, cy2, cz2 = _fps(cx1, cy1, cz1, 128)
    nexyz2 = jnp.stack([cx2, cy2, cz2], axis=-1)
    cent2 = jnp.pad(nexyz2, ((0, 0), (0, 0), (0, 13)))
    xt2 = jnp.pad(jnp.stack([cx1, cy1, cz1], axis=1),
                  ((0, 0), (0, 13), (0, 0)))
    w2 = _dist_bits(cent2, xt2, 64.0).reshape(B * 128, 512 // 16)

    l1 = _mlp(g1, _fold_weights(params, "sa1"),
              R=2048, g=32, cent=cent1.reshape(B * 512, 16))
    table2 = jnp.pad(
        jnp.concatenate([nexyz1, l1.reshape(B, 512, 64)], axis=-1),
        ((0, 0), (0, 0), (0, 13))).reshape(B * 512, 80)
    cent2s = jnp.pad(nexyz2, ((0, 0), (0, 0), (0, 77))).reshape(B * 128, 80)
    g2 = _sc_selgather(w2, table2, B=B, S=128, N=512, ns=16, dpad=80)
    l2 = _mlp(g2, _fold_weights(params, "sa2"),
              R=1024, g=16, cent=cent2s)

    rows3 = jnp.concatenate([nexyz2.reshape(B * 128, 3), l2], axis=-1)
    rows3 = jnp.pad(rows3, ((0, 0), (0, 384 - 259)))
    l3 = _mlp(rows3, _fold_weights(params, "sa3"), R=B * 128, g=128)
    return l3


# speedup vs baseline: 1.2920x; 1.2920x over previous
import functools

import jax
import jax.numpy as jnp
import numpy as np
from jax import lax
from jax.experimental import pallas as pl
from jax.experimental.pallas import tpu as pltpu
from jax.experimental.pallas import tpu_sc as plsc


def _fps_body(S, N, x_ref, y_ref, z_ref, idx_ref, cx_ref, cy_ref, cz_ref):
    B = x_ref.shape[0]
    x = x_ref[...]
    y = y_ref[...]
    z = z_ref[...]
    lane = lax.broadcasted_iota(jnp.int32, (B, N), 1)
    col = lax.broadcasted_iota(jnp.int32, (B, 128), 1)

    def step(j2, carry):
        dist, far, aidx, acx, acy, acz = carry
        oh = (lane == far).astype(jnp.float32)
        cx = jnp.sum(x * oh, axis=1, keepdims=True)
        cy = jnp.sum(y * oh, axis=1, keepdims=True)
        cz = jnp.sum(z * oh, axis=1, keepdims=True)
        sel = col == j2
        aidx = jnp.where(sel, jnp.broadcast_to(far, sel.shape), aidx)
        acx = jnp.where(sel, jnp.broadcast_to(cx, sel.shape), acx)
        acy = jnp.where(sel, jnp.broadcast_to(cy, sel.shape), acy)
        acz = jnp.where(sel, jnp.broadcast_to(cz, sel.shape), acz)
        xm = x - cx
        ym = y - cy
        zm = z - cz
        d = (xm * xm + ym * ym) + zm * zm
        dist = jnp.minimum(dist, d)
        m = jnp.max(dist, axis=1, keepdims=True)
        cand = jnp.where(dist == m, lane, N)
        far = jnp.min(cand, axis=1, keepdims=True).astype(jnp.int32)
        return dist, far, aidx, acx, acy, acz

    dist = jnp.full((B, N), 1e10, jnp.float32)
    far = jnp.zeros((B, 1), jnp.int32)
    for c in range(S // 128):
        sl = pl.ds(c * 128, 128)
        dist, far, aidx, acx, acy, acz = lax.fori_loop(
            0, 128, step,
            (dist, far, idx_ref[:, sl], cx_ref[:, sl], cy_ref[:, sl],
             cz_ref[:, sl]))
        idx_ref[:, sl] = aidx
        cx_ref[:, sl] = acx
        cy_ref[:, sl] = acy
        cz_ref[:, sl] = acz


def _fps(x, y, z, S):
    B, N = x.shape
    fn = pl.pallas_call(
        functools.partial(_fps_body, S, N),
        out_shape=(
            jax.ShapeDtypeStruct((B, S), jnp.int32),
            jax.ShapeDtypeStruct((B, S), jnp.float32),
            jax.ShapeDtypeStruct((B, S), jnp.float32),
            jax.ShapeDtypeStruct((B, S), jnp.float32),
        ),
    )
    return fn(x, y, z)


def _dist_body(r2, c_ref, xt_ref, p_ref, o_ref):
    c = c_ref[0]
    xt = xt_ref[0]
    sq_c = jnp.sum(c * c, axis=1, keepdims=True)
    x0 = xt[0:1, :]
    x1 = xt[1:2, :]
    x2 = xt[2:3, :]
    sq_x = (x0 * x0 + x1 * x1) + x2 * x2
    g = lax.dot_general(c, xt, (((1,), (0,)), ((), ())),
                        preferred_element_type=jnp.float32)
    d = (sq_c + sq_x) - 2.0 * g
    maskf = jnp.where(d <= r2, 1.0, 0.0).astype(jnp.bfloat16)
    bits = jnp.dot(maskf, p_ref[...], preferred_element_type=jnp.float32)
    o_ref[0] = bits.astype(jnp.int32)


def _dist_bits(cent_pad, xt_pad, r2):
    B, S, _ = cent_pad.shape
    N = xt_pad.shape[2]
    W = N // 16
    p = np.zeros((N, W), np.float32)
    p[np.arange(N), np.arange(N) // 16] = (2.0 ** (np.arange(N) % 16))
    fn = pl.pallas_call(
        functools.partial(_dist_body, r2),
        grid=(B,),
        in_specs=[
            pl.BlockSpec((1, S, 16), lambda b: (b, 0, 0)),
            pl.BlockSpec((1, 16, N), lambda b: (b, 0, 0)),
            pl.BlockSpec((N, W), lambda b: (0, 0)),
        ],
        out_specs=pl.BlockSpec((1, S, W), lambda b: (b, 0, 0)),
        out_shape=jax.ShapeDtypeStruct((B, S, W), jnp.int32),
    )
    return fn(cent_pad, xt_pad, jnp.asarray(p, jnp.bfloat16))


def _sc_selgather(bits, table, *, B, S, N, ns, dpad):
    rows = B * S
    W = N // 16
    info = plsc.get_sparse_core_info()
    nw = info.num_cores * info.num_subcores
    rpw = rows // nw
    sub = 8
    nouter = W // sub
    sb = 128 // ns
    nsb = rpw // sb
    mesh = plsc.VectorSubcoreMesh(core_axis_name="c", subcore_axis_name="s")

    @functools.partial(
        pl.kernel,
        out_type=jax.ShapeDtypeStruct((rows * ns, dpad), jnp.float32),
        mesh=mesh,
        compiler_params=pltpu.CompilerParams(needs_layout_passes=False,
                                             use_tc_tiling_on_sc=False),
        scratch_types=[
            pltpu.VMEM((rpw, W), jnp.int32),
            pltpu.VMEM((192,), jnp.int32),
            pltpu.VMEM((sb * ns,), jnp.int32),
            pltpu.VMEM((sb * ns, dpad), jnp.float32),
            pltpu.SMEM((1,), jnp.int32),
            pltpu.SemaphoreType.DMA,
        ],
    )
    def k(w_hbm, t_hbm, o_hbm, wslab, idxb, isel, rowsv, cnt_r, sem):
        wid = lax.axis_index("s") * info.num_cores + lax.axis_index("c")
        lanes = lax.broadcasted_iota(jnp.int32, (16,), 0)
        zeros = jnp.zeros((16,), jnp.int32)
        r0 = wid * rpw
        pltpu.sync_copy(w_hbm.at[pl.ds(r0, rpw)], wslab)

        def sb_body(i, carry):
            for rl in range(sb):
                rloc = i * sb + rl
                cnt_r[0] = 0

                def outer(c, carry2):
                    @pl.when(cnt_r[0] < ns)
                    def _():
                        cnt = cnt_r[0]
                        for j in range(sub):
                            wj = c * sub + j
                            w16 = plsc.load_gather(
                                wslab, [zeros + rloc, zeros + wj])
                            msk = ((w16 >> lanes) & 1) == 1
                            mi = msk.astype(jnp.int32)
                            pos = cnt + plsc.cumsum(mi) - 1
                            plsc.store_scatter(idxb, [pos], wj * 16 + lanes,
                                               mask=msk)
                            cnt = cnt + jnp.sum(mi)
                        cnt_r[0] = cnt
                    return carry2

                lax.fori_loop(0, nouter, outer, 0)
                cnt = cnt_r[0]
                first = plsc.load_gather(idxb, [zeros])
                base = ((r0 + rloc) // S) * N
                for kk in range(ns // 16):
                    slot = kk * 16 + lanes
                    cur = plsc.load_gather(idxb, [slot])
                    sel = jnp.where(slot < cnt, cur, first) + base
                    isel[rl * ns + kk * 16:rl * ns + (kk + 1) * 16] = sel

            pltpu.async_copy(t_hbm.at[isel], rowsv, sem).wait()
            pltpu.sync_copy(rowsv, o_hbm.at[pl.ds((r0 + i * sb) * ns,
                                                  sb * ns)])
            return carry

        lax.fori_loop(0, nsb, sb_body, 0)

    return k(bits, table)


def _mlp1_body(x_ref, c_ref, w1_ref, w1bd_ref, bt1_ref, w2bd_ref, bt2_ref,
               w3bd_ref, bt3_ref, o_ref):
    x = x_ref[...]
    q = jnp.dot(c_ref[...], w1_ref[...],
                preferred_element_type=jnp.float32)
    h = jnp.dot(x, w1bd_ref[...], preferred_element_type=jnp.float32)
    qt = jnp.concatenate([q] * 8, axis=1)
    bq = bt1_ref[...] - qt
    h = (h.reshape(256, 4, 256) + bq[:, None, :]).reshape(1024, 256)
    h = jnp.maximum(h, 0.0)
    h = jnp.maximum(jnp.dot(h, w2bd_ref[...],
                            preferred_element_type=jnp.float32)
                    + bt2_ref[...], 0.0)
    h = jnp.maximum(jnp.dot(h, w3bd_ref[...],
                            preferred_element_type=jnp.float32)
                    + bt3_ref[...], 0.0)
    m = jnp.max(h.reshape(256, 4, 512), axis=1)
    m = jnp.maximum(m[:, :256], m[:, 256:])
    m = jnp.maximum(m[:, :128], m[:, 128:])
    m = jnp.maximum(m[:, :64], m[:, 64:])
    o_ref[...] = m


def _mlp1_packed(xp, cent, wbs):
    (w1, b1), (w2, b2), (w3, b3) = wbs
    eye = jnp.eye(8, dtype=jnp.float32)
    args = [xp, cent, w1, jnp.kron(eye, w1), jnp.tile(b1, (1, 8)),
            jnp.kron(eye, w2), jnp.tile(b2, (1, 8)),
            jnp.kron(eye, w3), jnp.tile(b3, (1, 8))]
    in_specs = [pl.BlockSpec((1024, 128), lambda i: (i, 0)),
                pl.BlockSpec((256, 16), lambda i: (i, 0))]
    for a in args[2:]:
        in_specs.append(pl.BlockSpec(a.shape, lambda i: (0, 0)))
    fn = pl.pallas_call(
        _mlp1_body,
        grid=(xp.shape[0] // 1024,),
        in_specs=in_specs,
        out_specs=pl.BlockSpec((256, 64), lambda i: (i, 0)),
        out_shape=jax.ShapeDtypeStruct((xp.shape[0] // 4, 64), jnp.float32),
    )
    return fn(*args)


def _mlp_body(nl, g, has_cent, refs):
    x_ref = refs[0]
    o_ref = refs[-1]
    base = 1 + (1 if has_cent else 0)
    h = x_ref[...]
    R = h.shape[0]
    for l in range(nl):
        w = refs[base + 2 * l][...]
        b = refs[base + 2 * l + 1][...]
        h = jnp.dot(h, w, preferred_element_type=jnp.float32)
        if l == 0 and has_cent:
            q = jnp.dot(refs[1][...], w, preferred_element_type=jnp.float32)
            c0 = h.shape[1]
            h = h.reshape(R // g, g, c0) + (b - q)[:, None, :]
            h = h.reshape(R, c0)
        else:
            h = h + b
        h = jnp.maximum(h, 0.0)
    cout = h.shape[1]
    m = jnp.max(h.reshape(R // g, g, cout), axis=1)
    o_ref[...] = m


def _mlp(xrows, wbs, R, g, cent=None):
    rows, dpad = xrows.shape
    nl = len(wbs)
    cout = wbs[-1][0].shape[1]
    in_specs = [pl.BlockSpec((R, dpad), lambda i: (i, 0))]
    args = [xrows]
    if cent is not None:
        in_specs.append(pl.BlockSpec((R // g, dpad), lambda i: (i, 0)))
        args.append(cent)
    for (w, b) in wbs:
        in_specs.append(pl.BlockSpec(w.shape, lambda i: (0, 0)))
        in_specs.append(pl.BlockSpec(b.shape, lambda i: (0, 0)))
        args.extend([w, b])
    fn = pl.pallas_call(
        lambda *refs: _mlp_body(nl, g, cent is not None, refs),
        grid=(rows // R,),
        in_specs=in_specs,
        out_specs=pl.BlockSpec((R // g, cout), lambda i: (i, 0)),
        out_shape=jax.ShapeDtypeStruct((rows // g, cout), jnp.float32),
    )
    return fn(*args)


_LDIMS = {"sa1": [3, 32, 32, 64], "sa2": [67, 64, 128, 256],
          "sa3": [259, 256, 512, 1024]}
_PADIN = {"sa1": 16, "sa2": 128, "sa3": 384}


def _fold_weights(params, name):
    chs = _LDIMS[name]
    dpad = _PADIN[name]
    out = []
    c = np.float32(np.sqrt(1.0 + 1e-5))
    for l in range(len(chs) - 1):
        w = params[name + "_W" + str(l)]
        b = params[name + "_b" + str(l)]
        gm = params[name + "_g" + str(l)]
        be = params[name + "_be" + str(l)]
        s = gm / c
        wt = (w * s[:, None]).T
        bb = (b * s + be)[None, :]
        if l == 0 and wt.shape[0] != dpad:
            wt = jnp.pad(wt, ((0, dpad - wt.shape[0]), (0, 0)))
        out.append((wt, bb))
    return out


def kernel(xyz, params):
    B, _, N = xyz.shape

    x0, x1, x2 = xyz[:, 0, :], xyz[:, 1, :], xyz[:, 2, :]
    _, cx1, cy1, cz1 = _fps(x0, x1, x2, 512)
    nexyz1 = jnp.stack([cx1, cy1, cz1], axis=-1)
    cent1 = jnp.pad(nexyz1, ((0, 0), (0, 0), (0, 13)))
    xt1 = jnp.pad(xyz, ((0, 0), (0, 13), (0, 0)))
    w1 = _dist_bits(cent1, xt1, 16.0).reshape(B * 512, N // 16)
    table1 = jnp.pad(jnp.transpose(xyz, (0, 2, 1)),
                     ((0, 0), (0, 0), (0, 13))).reshape(B * N, 16)
    g1 = _sc_selgather(w1, table1, B=B, S=512, N=N, ns=32, dpad=16)

    _, cx2, cy2, cz2 = _fps(cx1, cy1, cz1, 128)
    nexyz2 = jnp.stack([cx2, cy2, cz2], axis=-1)
    cent2 = jnp.pad(nexyz2, ((0, 0), (0, 0), (0, 13)))
    xt2 = jnp.pad(jnp.stack([cx1, cy1, cz1], axis=1),
                  ((0, 0), (0, 13), (0, 0)))
    w2 = _dist_bits(cent2, xt2, 64.0).reshape(B * 128, 512 // 16)

    l1 = _mlp1_packed(g1.reshape(B * 512 * 32 * 16 // 128, 128),
                      cent1.reshape(B * 512, 16),
                      _fold_weights(params, "sa1"))
    table2 = jnp.pad(
        jnp.concatenate([nexyz1, l1.reshape(B, 512, 64)], axis=-1),
        ((0, 0), (0, 0), (0, 61))).reshape(B * 512, 128)
    cent2s = jnp.pad(nexyz2, ((0, 0), (0, 0), (0, 125))).reshape(B * 128, 128)
    g2 = _sc_selgather(w2, table2, B=B, S=128, N=512, ns=16, dpad=128)
    l2 = _mlp(g2, _fold_weights(params, "sa2"),
              R=1024, g=16, cent=cent2s)

    rows3 = jnp.concatenate([nexyz2.reshape(B * 128, 3), l2], axis=-1)
    rows3 = jnp.pad(rows3, ((0, 0), (0, 384 - 259)))
    l3 = _mlp(rows3, _fold_weights(params, "sa3"), R=B * 128, g=128)
    return l3
